# baseline (speedup 1.0000x reference)
import jax
import jax.numpy as jnp
from jax import lax
from jax.experimental import pallas as pl
from jax.experimental.pallas import tpu as pltpu
from jax.experimental.pallas import tpu_sc as plsc

NUM_FIELDS = 26
CARD = 100000
EMBED = 50
EPAD = 56

B, T = 1024, 20
N_ROWS = B * T * NUM_FIELDS

NC, NS = 2, 16
NW = NC * NS
CHUNK = 128
ROWS_PER_W = N_ROWS // NW
CHUNKS_PER_W = ROWS_PER_W // CHUNK


def _gather_body(w_hbm, g_hbm, out_hbm, idx_v, rows_v, sem):
    wid = lax.axis_index("s") * NC + lax.axis_index("c")
    w_base = wid * ROWS_PER_W

    def body(c, carry):
        base = w_base + c * CHUNK
        pltpu.sync_copy(g_hbm.at[pl.ds(base, CHUNK)], idx_v)
        pltpu.async_copy(w_hbm.at[idx_v], rows_v, sem).wait()
        pltpu.sync_copy(rows_v, out_hbm.at[pl.ds(base, CHUNK)])
        return carry

    lax.fori_loop(0, CHUNKS_PER_W, body, 0)


@jax.jit
def _gather(w_pad, g):
    mesh = plsc.VectorSubcoreMesh(core_axis_name="c", subcore_axis_name="s")
    return pl.kernel(
        _gather_body,
        out_type=jax.ShapeDtypeStruct((N_ROWS, EPAD), jnp.float32),
        mesh=mesh,
        scratch_types=[
            pltpu.VMEM((CHUNK,), jnp.int32),
            pltpu.VMEM((CHUNK, EPAD), jnp.float32),
            pltpu.SemaphoreType.DMA,
        ],
        compiler_params=pltpu.CompilerParams(use_tc_tiling_on_sc=False),
    )(w_pad, g)


def kernel(x, W):
    idx = x.reshape(B * T, NUM_FIELDS).astype(jnp.int32)
    g = (idx + jnp.arange(NUM_FIELDS, dtype=jnp.int32) * CARD).reshape(N_ROWS)
    w_pad = jnp.pad(
        W.reshape(NUM_FIELDS * CARD, EMBED), ((0, 0), (0, EPAD - EMBED))
    )
    out = _gather(w_pad, g)
    return out[:, :EMBED].reshape(B, T, NUM_FIELDS * EMBED, 1)
